# baseline (speedup 1.0000x reference)
import jax
import jax.numpy as jnp
from jax import lax
from jax.experimental import pallas as pl
from jax.experimental.pallas import tpu as pltpu
from jax.experimental.pallas import tpu_sc as plsc

B = 1024
L = 200
H = 768
EPS = 1e-12

NC = 2
NS = 16
NW = NC * NS
ROWS_PER_W = B // NW
PCHUNK = 40
NCHUNK = L // PCHUNK
NJ = H // 16

def _lane_sum(x):
    dnums = lax.GatherDimensionNumbers(
        offset_dims=(), collapsed_slice_dims=(0,), start_index_map=(0,))
    lanes = lax.iota(jnp.int32, 16)
    for k in (1, 2, 4, 8):
        idx = (lanes ^ k).reshape(16, 1)
        x = x + lax.gather(x, idx, dimension_numbers=dnums, slice_sizes=(1,),
                           mode=lax.GatherScatterMode.PROMISE_IN_BOUNDS)
    return x


def _body(ids_hbm, tt_hbm, word_hbm, pos_hbm, type_hbm, gamma_hbm, beta_hbm,
          out_hbm, idbuf, ttbuf, posbuf, wordbuf, typebuf, tdbuf, gbuf, bbuf,
          sem):
    wid = lax.axis_index("s") * NC + lax.axis_index("c")
    row0 = wid * ROWS_PER_W

    pltpu.sync_copy(type_hbm, typebuf)
    pltpu.sync_copy(gamma_hbm, gbuf)
    pltpu.sync_copy(beta_hbm, bbuf)
    pltpu.sync_copy(ids_hbm.at[pl.ds(row0 * L, ROWS_PER_W * L)], idbuf)
    pltpu.sync_copy(tt_hbm.at[pl.ds(row0 * L, ROWS_PER_W * L)],
                    ttbuf.at[pl.ds(0, ROWS_PER_W * L)])
    for j in range(NJ):
        sl = pl.ds(j * 16, 16)
        tdbuf[sl] = typebuf[1, sl] - typebuf[0, sl]

    for pc in range(NCHUNK):
        p0 = pc * PCHUNK
        pltpu.sync_copy(pos_hbm.at[pl.ds(p0, PCHUNK)], posbuf)

        def pp_body(k, c):
            for j in range(NJ):
                sl = pl.ds(j * 16, 16)
                posbuf[k, sl] = posbuf[k, sl] + typebuf[0, sl]
            return c
        lax.fori_loop(0, PCHUNK, pp_body, 0)

        def row_body(rb, c):
            b = row0 + rb
            base = rb * L + p0
            pltpu.async_copy(word_hbm.at[idbuf.at[pl.ds(base, PCHUNK)]],
                             wordbuf, sem).wait()

            def tok_body(i, c2):
                ttf = ttbuf[pl.ds(base + i, 16)][0].astype(jnp.float32)
                acc_s = jnp.zeros((16,), jnp.float32)
                acc_q = jnp.zeros((16,), jnp.float32)
                for j in range(NJ):
                    sl = pl.ds(j * 16, 16)
                    x = wordbuf[i, sl] + posbuf[i, sl] + ttf * tdbuf[sl]
                    wordbuf[i, sl] = x
                    acc_s = acc_s + x
                    acc_q = acc_q + x * x
                mean = _lane_sum(acc_s) * (1.0 / H)
                var = _lane_sum(acc_q) * (1.0 / H) - mean * mean
                t = var + EPS
                ti = lax.bitcast_convert_type(t, jnp.int32)
                yi = jnp.int32(0x5F3759DF) - (ti >> 1)
                y = lax.bitcast_convert_type(yi, jnp.float32)
                for _ in range(3):
                    y = y * (1.5 - 0.5 * t * y * y)
                rstd = y
                m = mean * rstd
                for j in range(NJ):
                    sl = pl.ds(j * 16, 16)
                    x = wordbuf[i, sl]
                    wordbuf[i, sl] = (x * rstd - m) * gbuf[sl] + bbuf[sl]
                return c2
            lax.fori_loop(0, PCHUNK, tok_body, 0)

            pltpu.sync_copy(wordbuf, out_hbm.at[b, pl.ds(p0, PCHUNK)])
            return c
        lax.fori_loop(0, ROWS_PER_W, row_body, 0)


def kernel(input_ids, token_type_ids, word_emb, pos_emb, type_emb, ln_gamma,
           ln_beta):
    mesh = plsc.VectorSubcoreMesh(core_axis_name="c", subcore_axis_name="s")
    f = pl.kernel(
        _body,
        out_type=jax.ShapeDtypeStruct((B, L, H), jnp.float32),
        mesh=mesh,
        scratch_types=[
            pltpu.VMEM((ROWS_PER_W * L,), jnp.int32),
            pltpu.VMEM((ROWS_PER_W * L + 16,), jnp.int32),
            pltpu.VMEM((PCHUNK, H), jnp.float32),
            pltpu.VMEM((PCHUNK, H), jnp.float32),
            pltpu.VMEM((2, H), jnp.float32),
            pltpu.VMEM((H,), jnp.float32),
            pltpu.VMEM((H,), jnp.float32),
            pltpu.VMEM((H,), jnp.float32),
            pltpu.SemaphoreType.DMA,
        ],
    )
    return f(input_ids.reshape(B * L), token_type_ids.reshape(B * L),
             word_emb, pos_emb, type_emb, ln_gamma, ln_beta)


# speedup vs baseline: 4.6955x; 4.6955x over previous
import jax
import jax.numpy as jnp
from jax import lax
from jax.experimental import pallas as pl
from jax.experimental.pallas import tpu as pltpu
from jax.experimental.pallas import tpu_sc as plsc

B = 1024
L = 200
H = 768
EPS = 1e-12

NC = 2
NS = 16
NW = NC * NS
TOK = B * L
TOK_W = TOK // NW
GCHUNK = 32
NG = TOK_W // GCHUNK
NSLOT = 4

RB = 8


def _gather_body(ids_hbm, word_hbm, g_hbm, idbuf, slots, gsem, wbsem):
    wid = lax.axis_index("s") * NC + lax.axis_index("c")
    w0 = wid * TOK_W
    pltpu.sync_copy(ids_hbm.at[pl.ds(w0, TOK_W)], idbuf)

    def gather(c):
        return pltpu.make_async_copy(
            word_hbm.at[idbuf.at[pl.ds(c * GCHUNK, GCHUNK)]],
            slots.at[lax.rem(c, NSLOT)], gsem)

    def writeback(c):
        return pltpu.make_async_copy(
            slots.at[lax.rem(c, NSLOT)],
            g_hbm.at[pl.ds(w0 + c * GCHUNK, GCHUNK)], wbsem)

    def step(c, carry):
        @pl.when(c < NG)
        def _():
            @pl.when(c >= NSLOT)
            def _():
                writeback(c - NSLOT).wait()
            gather(c).start()

        @pl.when(jnp.logical_and(c >= 2, c - 2 < NG))
        def _():
            gather(c - 2).wait()
            writeback(c - 2).start()

        return carry

    lax.fori_loop(0, NG + 2, step, 0)
    for k in range(NSLOT):
        writeback(NG - NSLOT + k).wait()


def _ln_body(tt_ref, g_ref, pos_ref, type_ref, gam_ref, bet_ref, out_ref):
    x = g_ref[...]
    tt = tt_ref[...]
    tsel = jnp.where(tt == 1, type_ref[1], type_ref[0])
    x = x + pos_ref[...][None, :, :] + tsel
    mean = jnp.mean(x, axis=-1, keepdims=True)
    xc = x - mean
    var = jnp.mean(xc * xc, axis=-1, keepdims=True)
    y = xc * lax.rsqrt(var + EPS)
    out_ref[...] = y * gam_ref[...] + bet_ref[...]


def kernel(input_ids, token_type_ids, word_emb, pos_emb, type_emb, ln_gamma,
           ln_beta):
    mesh = plsc.VectorSubcoreMesh(core_axis_name="c", subcore_axis_name="s")
    gather = pl.kernel(
        _gather_body,
        out_type=jax.ShapeDtypeStruct((TOK, H), jnp.float32),
        mesh=mesh,
        scratch_types=[
            pltpu.VMEM((TOK_W,), jnp.int32),
            pltpu.VMEM((NSLOT, GCHUNK, H), jnp.float32),
            pltpu.SemaphoreType.DMA,
            pltpu.SemaphoreType.DMA,
        ],
    )
    g = gather(input_ids.reshape(TOK), word_emb)

    out = pl.pallas_call(
        _ln_body,
        grid=(B // RB,),
        in_specs=[
            pl.BlockSpec((RB, L, 1), lambda i: (i, 0, 0)),
            pl.BlockSpec((RB, L, H), lambda i: (i, 0, 0)),
            pl.BlockSpec((L, H), lambda i: (0, 0)),
            pl.BlockSpec((2, H), lambda i: (0, 0)),
            pl.BlockSpec((H,), lambda i: (0,)),
            pl.BlockSpec((H,), lambda i: (0,)),
        ],
        out_specs=pl.BlockSpec((RB, L, H), lambda i: (i, 0, 0)),
        out_shape=jax.ShapeDtypeStruct((B, L, H), jnp.float32),
    )(token_type_ids.reshape(B, L, 1), g.reshape(B, L, H), pos_emb[:L],
      type_emb, ln_gamma, ln_beta)
    return out
